# baseline (speedup 1.0000x reference)
import functools

import jax
import jax.numpy as jnp
from jax import lax
from jax.experimental import pallas as pl
from jax.experimental.pallas import tpu as pltpu
from jax.experimental.pallas import tpu_sc as plsc

VOCAB = 100000
EMB = 128
HID = 1024
B, L = 4, 2048
PAD = 8
PADL = L + 2 * PAD
NW = 32
CH = (B * L) // NW
TL = L


def _sc_gather(ids_flat, table):
    mesh = plsc.VectorSubcoreMesh(core_axis_name="c", subcore_axis_name="s")

    @functools.partial(
        pl.kernel,
        mesh=mesh,
        out_type=jax.ShapeDtypeStruct((B * PADL, EMB), jnp.float32),
        scratch_types=[
            pltpu.VMEM((CH,), jnp.int32),
            pltpu.VMEM((CH, EMB), jnp.float32),
            pltpu.VMEM((PAD, EMB), jnp.float32),
            pltpu.SemaphoreType.DMA,
            pltpu.SemaphoreType.DMA,
            pltpu.SemaphoreType.DMA,
        ],
    )
    def gather_kernel(idx_hbm, table_hbm, out_hbm, idx_v, rows_v, zero_v,
                      sem, sem2, zsem):
        cid = lax.axis_index("c")
        sid = lax.axis_index("s")
        wid = cid * 16 + sid
        fb = wid * CH
        b = fb // L
        out_row = b * PADL + PAD + (fb - b * L)
        z = jnp.zeros((16,), jnp.float32)
        for i in range(PAD):
            for j in range(EMB // 16):
                zero_v[i, pl.ds(j * 16, 16)] = z
        zb = wid // 2
        zrow = zb * PADL + (wid % 2) * (PAD + L)
        is_zw = wid < 2 * B

        @pl.when(is_zw)
        def _():
            pltpu.async_copy(zero_v, out_hbm.at[pl.ds(zrow, PAD)], zsem)

        H2 = CH // 2
        pltpu.sync_copy(idx_hbm.at[pl.ds(fb, CH)], idx_v)
        c0 = pltpu.async_copy(table_hbm.at[idx_v.at[pl.ds(0, H2)]],
                              rows_v.at[pl.ds(0, H2)], sem)
        c1 = pltpu.async_copy(table_hbm.at[idx_v.at[pl.ds(H2, H2)]],
                              rows_v.at[pl.ds(H2, H2)], sem2)
        c0.wait()
        pltpu.sync_copy(rows_v.at[pl.ds(0, H2)], out_hbm.at[pl.ds(out_row, H2)])
        c1.wait()
        pltpu.sync_copy(rows_v.at[pl.ds(H2, H2)],
                        out_hbm.at[pl.ds(out_row + H2, H2)])

        @pl.when(is_zw)
        def _():
            pltpu.make_async_copy(zero_v, out_hbm.at[pl.ds(zrow, PAD)],
                                  zsem).wait()

    return gather_kernel(ids_flat, table)


def _tc_body(epad_ref, w_ref, pos_ref, lnw_ref, c0_ref, out_ref, acc_ref):
    @pl.when(pl.program_id(0) == 0)
    def _():
        acc_ref[...] = pos_ref[...] * lnw_ref[...] + c0_ref[...]

    ec = epad_ref[0, pl.ds(PAD, TL), :].astype(jnp.bfloat16)
    el = epad_ref[0, pl.ds(PAD + 1, TL), :].astype(jnp.bfloat16)
    er = epad_ref[0, pl.ds(PAD - 1, TL), :].astype(jnp.bfloat16)
    tri = jnp.concatenate([el, ec, er], axis=1)
    h = jnp.dot(tri, w_ref[...], preferred_element_type=jnp.float32)
    out_ref[0] = h + acc_ref[...]


def kernel(input_ids, word_embeddings, dense_W, dense_b, pos_emb, type_emb,
           ln_weight, ln_bias):
    ids_flat = input_ids.reshape(-1).astype(jnp.int32)
    epad = _sc_gather(ids_flat, word_embeddings)
    epad = epad.reshape(B, PADL, EMB)

    grid = (B,)
    out = pl.pallas_call(
        _tc_body,
        grid=grid,
        in_specs=[
            pl.BlockSpec((1, PADL, EMB), lambda b: (b, 0, 0)),
            pl.BlockSpec((3 * EMB, HID), lambda b: (0, 0)),
            pl.BlockSpec((TL, HID), lambda b: (0, 0)),
            pl.BlockSpec((1, HID), lambda b: (0, 0)),
            pl.BlockSpec((1, HID), lambda b: (0, 0)),
        ],
        out_specs=pl.BlockSpec((1, TL, HID), lambda b: (b, 0, 0)),
        out_shape=jax.ShapeDtypeStruct((B, L, HID), jnp.float32),
        scratch_shapes=[pltpu.VMEM((TL, HID), jnp.float32)],
    )(
        epad,
        (dense_W * ln_weight[None, :]).astype(jnp.bfloat16),
        pos_emb,
        ln_weight.reshape(1, HID),
        ((dense_b + type_emb[0]) * ln_weight + ln_bias).reshape(1, HID),
    )
    return out


# speedup vs baseline: 1.0055x; 1.0055x over previous
import functools

import jax
import jax.numpy as jnp
from jax import lax
from jax.experimental import pallas as pl
from jax.experimental.pallas import tpu as pltpu
from jax.experimental.pallas import tpu_sc as plsc

VOCAB = 100000
EMB = 128
HID = 1024
B, L = 4, 2048
PAD = 8
PADL = L + 2 * PAD
NW = 32
CH = (B * L) // NW
TL = L


def _sc_gather(ids_flat, table):
    mesh = plsc.VectorSubcoreMesh(core_axis_name="c", subcore_axis_name="s")

    @functools.partial(
        pl.kernel,
        mesh=mesh,
        out_type=jax.ShapeDtypeStruct((B * PADL, EMB), jnp.float32),
        scratch_types=[
            pltpu.VMEM((CH,), jnp.int32),
            pltpu.VMEM((CH, EMB), jnp.float32),
            pltpu.VMEM((PAD, EMB), jnp.float32),
            pltpu.SemaphoreType.DMA,
            pltpu.SemaphoreType.DMA,
        ],
    )
    def gather_kernel(idx_hbm, table_hbm, out_hbm, idx_v, rows_v, zero_v,
                      sem, sem2):
        cid = lax.axis_index("c")
        sid = lax.axis_index("s")
        wid = cid * 16 + sid
        fb = wid * CH
        b = fb // L
        out_row = b * PADL + PAD + (fb - b * L)
        H2 = CH // 2
        pltpu.sync_copy(idx_hbm.at[pl.ds(fb, CH)], idx_v)
        c0 = pltpu.async_copy(table_hbm.at[idx_v.at[pl.ds(0, H2)]],
                              rows_v.at[pl.ds(0, H2)], sem)
        c1 = pltpu.async_copy(table_hbm.at[idx_v.at[pl.ds(H2, H2)]],
                              rows_v.at[pl.ds(H2, H2)], sem2)
        c0.wait()
        pltpu.sync_copy(rows_v.at[pl.ds(0, H2)], out_hbm.at[pl.ds(out_row, H2)])
        c1.wait()
        pltpu.sync_copy(rows_v.at[pl.ds(H2, H2)],
                        out_hbm.at[pl.ds(out_row + H2, H2)])
        z = jnp.zeros((16,), jnp.float32)
        for i in range(PAD):
            for j in range(EMB // 16):
                zero_v[i, pl.ds(j * 16, 16)] = z
        zb = wid // 2
        zrow = zb * PADL + (wid % 2) * (PAD + L)

        @pl.when(wid < 2 * B)
        def _():
            pltpu.sync_copy(zero_v, out_hbm.at[pl.ds(zrow, PAD)])

    return gather_kernel(ids_flat, table)


def _tc_body(epad_ref, w_ref, pos_ref, lnw_ref, c0_ref, out_ref, acc_ref):
    @pl.when(pl.program_id(0) == 0)
    def _():
        acc_ref[...] = pos_ref[...] * lnw_ref[...] + c0_ref[...]

    ec = epad_ref[0, pl.ds(PAD, TL), :].astype(jnp.bfloat16)
    el = epad_ref[0, pl.ds(PAD + 1, TL), :].astype(jnp.bfloat16)
    er = epad_ref[0, pl.ds(PAD - 1, TL), :].astype(jnp.bfloat16)
    tri = jnp.concatenate([el, ec, er], axis=1)
    h = jnp.dot(tri, w_ref[...], preferred_element_type=jnp.float32)
    out_ref[0] = h + acc_ref[...]


def kernel(input_ids, word_embeddings, dense_W, dense_b, pos_emb, type_emb,
           ln_weight, ln_bias):
    ids_flat = input_ids.reshape(-1).astype(jnp.int32)
    epad = _sc_gather(ids_flat, word_embeddings)
    epad = epad.reshape(B, PADL, EMB)

    grid = (B,)
    out = pl.pallas_call(
        _tc_body,
        grid=grid,
        in_specs=[
            pl.BlockSpec((1, PADL, EMB), lambda b: (b, 0, 0)),
            pl.BlockSpec((3 * EMB, HID), lambda b: (0, 0)),
            pl.BlockSpec((TL, HID), lambda b: (0, 0)),
            pl.BlockSpec((1, HID), lambda b: (0, 0)),
            pl.BlockSpec((1, HID), lambda b: (0, 0)),
        ],
        out_specs=pl.BlockSpec((1, TL, HID), lambda b: (b, 0, 0)),
        out_shape=jax.ShapeDtypeStruct((B, L, HID), jnp.float32),
        scratch_shapes=[pltpu.VMEM((TL, HID), jnp.float32)],
    )(
        epad,
        (dense_W * ln_weight[None, :]).astype(jnp.bfloat16),
        pos_emb,
        ln_weight.reshape(1, HID),
        ((dense_b + type_emb[0]) * ln_weight + ln_bias).reshape(1, HID),
    )
    return out
